# baseline (speedup 1.0000x reference)
import jax
import jax.numpy as jnp
from jax.experimental import pallas as pl
from jax.experimental.pallas import tpu as pltpu


def _copy_body(x_hbm, o_hbm, sem):
    copy = pltpu.make_async_copy(x_hbm, o_hbm, sem)
    copy.start()
    copy.wait()


def kernel(method_soft, label, features):
    del label, features
    return pl.pallas_call(
        _copy_body,
        in_specs=[pl.BlockSpec(memory_space=pl.ANY)],
        out_specs=pl.BlockSpec(memory_space=pl.ANY),
        scratch_shapes=[pltpu.SemaphoreType.DMA],
        out_shape=jax.ShapeDtypeStruct(method_soft.shape, method_soft.dtype),
    )(method_soft)


# speedup vs baseline: 12.3014x; 12.3014x over previous
import jax
import jax.numpy as jnp
from jax.experimental import pallas as pl
from jax.experimental.pallas import tpu as pltpu

_ROWS = 16384
_BLOCKS = 8
_BLOCK_ROWS = _ROWS // _BLOCKS


def _copy_body(x_ref, o_ref):
    o_ref[...] = x_ref[...]


def kernel(method_soft, label, features):
    del label, features
    return pl.pallas_call(
        _copy_body,
        grid=(_BLOCKS,),
        in_specs=[pl.BlockSpec((_BLOCK_ROWS, 20), lambda i: (i, 0))],
        out_specs=pl.BlockSpec((_BLOCK_ROWS, 20), lambda i: (i, 0)),
        out_shape=jax.ShapeDtypeStruct(method_soft.shape, method_soft.dtype),
    )(method_soft)


# speedup vs baseline: 13.5714x; 1.1032x over previous
import jax
import jax.numpy as jnp
from jax.experimental import pallas as pl
from jax.experimental.pallas import tpu as pltpu

_ROWS = 16384
_BLOCKS = 8
_BLOCK_ROWS = _ROWS // _BLOCKS


def _copy_body(x_ref, o_ref):
    o_ref[...] = x_ref[...]


def kernel(method_soft, label, features):
    del label, features
    return pl.pallas_call(
        _copy_body,
        out_shape=jax.ShapeDtypeStruct(method_soft.shape, method_soft.dtype),
    )(method_soft)
